# Initial kernel scaffold; baseline (speedup 1.0000x reference)
import jax
import jax.numpy as jnp
from jax.experimental import pallas as pl


def kernel(x, edge_index, edge_attr, batch, y, params):
    raise NotImplementedError("write your pallas kernel here")


# speedup vs baseline: 7.7375x; 7.7375x over previous
import functools

import jax
import jax.numpy as jnp
from jax import lax
from jax.experimental import pallas as pl
from jax.experimental.pallas import tpu as pltpu
from jax.experimental.pallas import tpu_sc as plsc

N = 10000
E = 320000
D = 128
R = 4
NUM_LAYERS = 3
B = 64

NC = 2
NS = 16
NW = NC * NS
LANES = 16

CH = 80
EW = E // NW
NCH = EW // CH
RT = 632
NPAD = NS * RT


def _sc_scatter_body(h_hbm, src_hbm, ety_hbm, dst_hbm, zeros_hbm, out_hbm,
                gidxb, dstb, rows, acc, sem):
    cid = lax.axis_index("c")
    sid = lax.axis_index("s")
    wid = cid * NS + sid

    pltpu.sync_copy(zeros_hbm.at[pl.ds(sid * RT, RT)],
                    acc.at[pl.ds(sid * RT, RT)])

    pltpu.sync_copy(src_hbm.at[wid], gidxb)
    pltpu.sync_copy(ety_hbm.at[wid], dstb)

    def gix_body(ch, _):
        for j in range(CH // LANES):
            s = gidxb[ch, pl.ds(j * LANES, LANES)]
            t = dstb[ch, pl.ds(j * LANES, LANES)]
            gidxb[ch, pl.ds(j * LANES, LANES)] = s * R + t
        return 0
    lax.fori_loop(0, NCH, gix_body, 0)
    pltpu.sync_copy(dst_hbm.at[wid], dstb)

    plsc.subcore_barrier()

    def chunk_body(ch, _):
        pltpu.async_copy(h_hbm.at[gidxb.at[ch]], rows, sem).wait()
        pltpu.sync_copy(rows, acc.at[dstb.at[ch]], add=True)
        return 0
    lax.fori_loop(0, NCH, chunk_body, 0)

    plsc.subcore_barrier()
    pltpu.sync_copy(acc.at[pl.ds(sid * RT, RT)],
                    out_hbm.at[pl.ds(cid * NPAD + sid * RT, RT)])


_sc_scatter_cache = []


def _sc_scatter(hflat, src, ety, dst, zeros):
    if not _sc_scatter_cache:
        mesh = plsc.VectorSubcoreMesh(core_axis_name="c",
                                      subcore_axis_name="s")
        _sc_scatter_cache.append(pl.kernel(
            _sc_scatter_body,
            out_type=jax.ShapeDtypeStruct((NC * NPAD, D), jnp.float32),
            mesh=mesh,
            scratch_types=[
                pltpu.VMEM((NCH, CH), jnp.int32),
                pltpu.VMEM((NCH, CH), jnp.int32),
                pltpu.VMEM((CH, D), jnp.float32),
                pltpu.VMEM_SHARED((NPAD, D), jnp.float32),
                pltpu.SemaphoreType.DMA,
            ],
        ))
    return _sc_scatter_cache[0](hflat, src, ety, dst, zeros)


_MB = 2000


def _mm512_body(h_ref, w_ref, o_ref):
    o_ref[...] = jnp.dot(h_ref[...], w_ref[...],
                         preferred_element_type=jnp.float32)


def _mm512(h, wcat):
    return pl.pallas_call(
        _mm512_body,
        grid=(N // _MB,),
        in_specs=[
            pl.BlockSpec((_MB, D), lambda i: (i, 0)),
            pl.BlockSpec((D, R * D), lambda i: (0, 0)),
        ],
        out_specs=pl.BlockSpec((_MB, R * D), lambda i: (i, 0)),
        out_shape=jax.ShapeDtypeStruct((N, R * D), jnp.float32),
    )(h, wcat)


def _combine_body(relu, h_ref, a0_ref, a1_ref, w_ref, b_ref, o_ref):
    z = (jnp.dot(h_ref[...], w_ref[...], preferred_element_type=jnp.float32)
         + a0_ref[...] + a1_ref[...] + b_ref[...])
    if relu:
        z = jnp.maximum(z, 0.0)
    o_ref[...] = z


def _combine(h, agg0, agg1, wroot, bias, relu):
    return pl.pallas_call(
        functools.partial(_combine_body, relu),
        grid=(N // _MB,),
        in_specs=[
            pl.BlockSpec((_MB, D), lambda i: (i, 0)),
            pl.BlockSpec((_MB, D), lambda i: (i, 0)),
            pl.BlockSpec((_MB, D), lambda i: (i, 0)),
            pl.BlockSpec((D, D), lambda i: (0, 0)),
            pl.BlockSpec((1, D), lambda i: (0, 0)),
        ],
        out_specs=pl.BlockSpec((_MB, D), lambda i: (i, 0)),
        out_shape=jax.ShapeDtypeStruct((N, D), jnp.float32),
    )(h, agg0, agg1, wroot, bias)


_PB = 2000
_NHEAD = 7
_RELU_AFTER = (True, False, False, True, True, True, False)


def _pool_head_body(batch_ref, h_ref, w_ref, b_ref, y_ref,
                    out_ref, loss_ref, pooled_ref):
    pid = pl.program_id(0)

    @pl.when(pid == 0)
    def _init():
        pooled_ref[...] = jnp.zeros_like(pooled_ref)

    bb = batch_ref[...].reshape(1, _PB)
    seg = jax.lax.broadcasted_iota(jnp.int32, (B, _PB), 0)
    onehot = (seg == bb).astype(jnp.float32)
    pooled_ref[...] += jnp.dot(onehot, h_ref[...],
                               preferred_element_type=jnp.float32)

    @pl.when(pid == (N // _PB) - 1)
    def _head():
        z = pooled_ref[...]
        for j in range(_NHEAD):
            z = jnp.dot(z, w_ref[j], preferred_element_type=jnp.float32) \
                + b_ref[j]
            if _RELU_AFTER[j]:
                z = jnp.maximum(z, 0.0)
        out_ref[...] = z
        d = z[:, 0:1] - y_ref[...]
        loss_ref[...] = jnp.sqrt(jnp.sum(d * d) * (1.0 / B)).reshape(1, 1)


def _pool_head(batch3, h, wstack, bstack, y):
    return pl.pallas_call(
        _pool_head_body,
        grid=(N // _PB,),
        in_specs=[
            pl.BlockSpec((1, 1, _PB), lambda i: (i, 0, 0)),
            pl.BlockSpec((_PB, D), lambda i: (i, 0)),
            pl.BlockSpec((_NHEAD, D, D), lambda i: (0, 0, 0)),
            pl.BlockSpec((_NHEAD, 1, D), lambda i: (0, 0, 0)),
            pl.BlockSpec((B, 1), lambda i: (0, 0)),
        ],
        out_specs=[
            pl.BlockSpec((B, D), lambda i: (0, 0)),
            pl.BlockSpec((1, 1), lambda i: (0, 0)),
        ],
        out_shape=[
            jax.ShapeDtypeStruct((B, D), jnp.float32),
            jax.ShapeDtypeStruct((1, 1), jnp.float32),
        ],
        scratch_shapes=[pltpu.VMEM((B, D), jnp.float32)],
    )(batch3, h, wstack, bstack, y)


def _pad_w(w):
    out = jnp.zeros((D, D), jnp.float32)
    return out.at[: w.shape[0], : w.shape[1]].set(w)


def _pad_b(b):
    out = jnp.zeros((1, D), jnp.float32)
    return out.at[0, : b.shape[0]].set(b)


def kernel(x, edge_index, edge_attr, batch, y, params):
    src = edge_index[0].reshape(NW, NCH, CH).astype(jnp.int32)
    ety = edge_attr.reshape(NW, NCH, CH).astype(jnp.int32)
    dst = edge_index[1].reshape(NW, NCH, CH).astype(jnp.int32)
    zeros = jnp.zeros((NPAD, D), jnp.float32)

    h = x
    for i in range(NUM_LAYERS):
        wrel = params["conv%d_Wrel" % i]
        wcat = wrel.transpose(1, 0, 2).reshape(D, R * D)
        hflat = _mm512(h, wcat).reshape(N * R, D)
        parts = _sc_scatter(hflat, src, ety, dst, zeros)
        agg0 = parts[:N]
        agg1 = parts[NPAD:NPAD + N]
        bias = params["conv%d_b" % i].reshape(1, D)
        h = _combine(h, agg0, agg1, params["conv%d_Wroot" % i], bias,
                     relu=(i != NUM_LAYERS - 1))

    wstack = jnp.stack([
        _pad_w(params["fc1_W"]), _pad_w(params["fc2_W"]),
        _pad_w(params["first_W"]),
        _pad_w(params["mlp0_W"]), _pad_w(params["mlp1_W"]),
        _pad_w(params["mlp2_W"]), _pad_w(params["mlp3_W"]),
    ])
    bstack = jnp.stack([
        _pad_b(params["fc1_b"]), _pad_b(params["fc2_b"]),
        _pad_b(params["first_b"]),
        _pad_b(params["mlp0_b"]), _pad_b(params["mlp1_b"]),
        _pad_b(params["mlp2_b"]), _pad_b(params["mlp3_b"]),
    ])
    batch3 = batch.astype(jnp.int32).reshape(N // _PB, 1, _PB)

    out_pad, loss2 = _pool_head(batch3, h, wstack, bstack, y)
    out = out_pad[:, :1]
    loss = loss2[0, 0]
    return (out, loss, loss)


# speedup vs baseline: 9.6107x; 1.2421x over previous
import functools

import jax
import jax.numpy as jnp
from jax import lax
from jax.experimental import pallas as pl
from jax.experimental.pallas import tpu as pltpu
from jax.experimental.pallas import tpu_sc as plsc

N = 10000
E = 320000
D = 128
R = 4
NUM_LAYERS = 3
B = 64

NC = 2
NS = 16
NW = NC * NS
LANES = 16

CH = 80
EW = E // NW
NCH = EW // CH
RT = 632
NPAD = NS * RT


def _sc_scatter_body(h_hbm, gidx_hbm, dst_hbm, zeros_hbm, out_hbm,
                     gidxb, dstb, rows0, rows1, acc, sem0, sem1):
    cid = lax.axis_index("c")
    sid = lax.axis_index("s")
    wid = cid * NS + sid

    pltpu.sync_copy(zeros_hbm.at[pl.ds(sid * RT, RT)],
                    acc.at[pl.ds(sid * RT, RT)])

    pltpu.sync_copy(gidx_hbm.at[pl.ds(wid * EW, EW)], gidxb)
    pltpu.sync_copy(dst_hbm.at[wid], dstb)

    plsc.subcore_barrier()

    def start_gather(ch, buf, sem):
        return pltpu.async_copy(
            h_hbm.at[gidxb.at[pl.ds(ch * CH, CH)]], buf, sem)

    start_gather(0, rows0, sem0)

    def pair_body(k, _):
        ch = 2 * k
        pltpu.make_async_copy(h_hbm.at[gidxb.at[pl.ds(ch * CH, CH)]],
                              rows0, sem0).wait()
        start_gather(ch + 1, rows1, sem1)
        pltpu.sync_copy(rows0, acc.at[dstb.at[ch]], add=True)
        pltpu.make_async_copy(h_hbm.at[gidxb.at[pl.ds((ch + 1) * CH, CH)]],
                              rows1, sem1).wait()
        start_gather(ch + 2, rows0, sem0)
        pltpu.sync_copy(rows1, acc.at[dstb.at[ch + 1]], add=True)
        return 0
    lax.fori_loop(0, (NCH - 1) // 2, pair_body, 0)

    ch_last = NCH - 1
    pltpu.make_async_copy(h_hbm.at[gidxb.at[pl.ds(ch_last * CH, CH)]],
                          rows0, sem0).wait()
    pltpu.sync_copy(rows0, acc.at[dstb.at[ch_last]], add=True)

    plsc.subcore_barrier()
    pltpu.sync_copy(acc.at[pl.ds(sid * RT, RT)],
                    out_hbm.at[pl.ds(cid * NPAD + sid * RT, RT)])


_sc_scatter_cache = []


def _sc_scatter(hflat, gidx, dst, zeros):
    if not _sc_scatter_cache:
        mesh = plsc.VectorSubcoreMesh(core_axis_name="c",
                                      subcore_axis_name="s")
        _sc_scatter_cache.append(pl.kernel(
            _sc_scatter_body,
            out_type=jax.ShapeDtypeStruct((NC * NPAD, D), jnp.float32),
            mesh=mesh,
            scratch_types=[
                pltpu.VMEM((EW,), jnp.int32),
                pltpu.VMEM((NCH, CH), jnp.int32),
                pltpu.VMEM((CH, D), jnp.float32),
                pltpu.VMEM((CH, D), jnp.float32),
                pltpu.VMEM_SHARED((NPAD, D), jnp.float32),
                pltpu.SemaphoreType.DMA,
                pltpu.SemaphoreType.DMA,
            ],
        ))
    return _sc_scatter_cache[0](hflat, gidx, dst, zeros)


_MB = 2000


def _gidx_body(s_ref, t_ref, o_ref):
    o_ref[...] = s_ref[...] * R + t_ref[...]


def _gidx(src2, ety2):
    return pl.pallas_call(
        _gidx_body,
        out_shape=jax.ShapeDtypeStruct((E // D, D), jnp.int32),
    )(src2, ety2)


def _mm512_body(h_ref, w_ref, o_ref):
    o_ref[...] = jnp.dot(h_ref[...], w_ref[...],
                         preferred_element_type=jnp.float32)


def _mm512(h, wcat):
    return pl.pallas_call(
        _mm512_body,
        grid=(N // _MB,),
        in_specs=[
            pl.BlockSpec((_MB, D), lambda i: (i, 0)),
            pl.BlockSpec((D, R * D), lambda i: (0, 0)),
        ],
        out_specs=pl.BlockSpec((_MB, R * D), lambda i: (i, 0)),
        out_shape=jax.ShapeDtypeStruct((N, R * D), jnp.float32),
    )(h, wcat)


def _combine_body(relu, h_ref, a0_ref, a1_ref, w_ref, b_ref, o_ref):
    z = (jnp.dot(h_ref[...], w_ref[...], preferred_element_type=jnp.float32)
         + a0_ref[...] + a1_ref[...] + b_ref[...])
    if relu:
        z = jnp.maximum(z, 0.0)
    o_ref[...] = z


def _combine(h, agg0, agg1, wroot, bias, relu):
    return pl.pallas_call(
        functools.partial(_combine_body, relu),
        grid=(N // _MB,),
        in_specs=[
            pl.BlockSpec((_MB, D), lambda i: (i, 0)),
            pl.BlockSpec((_MB, D), lambda i: (i, 0)),
            pl.BlockSpec((_MB, D), lambda i: (i, 0)),
            pl.BlockSpec((D, D), lambda i: (0, 0)),
            pl.BlockSpec((1, D), lambda i: (0, 0)),
        ],
        out_specs=pl.BlockSpec((_MB, D), lambda i: (i, 0)),
        out_shape=jax.ShapeDtypeStruct((N, D), jnp.float32),
    )(h, agg0, agg1, wroot, bias)


_PB = 2000
_NHEAD = 7
_RELU_AFTER = (True, False, False, True, True, True, False)


def _pool_head_body(batch_ref, h_ref, w_ref, b_ref, y_ref,
                    out_ref, loss_ref, pooled_ref):
    pid = pl.program_id(0)

    @pl.when(pid == 0)
    def _init():
        pooled_ref[...] = jnp.zeros_like(pooled_ref)

    bb = batch_ref[...].reshape(1, _PB)
    seg = jax.lax.broadcasted_iota(jnp.int32, (B, _PB), 0)
    onehot = (seg == bb).astype(jnp.float32)
    pooled_ref[...] += jnp.dot(onehot, h_ref[...],
                               preferred_element_type=jnp.float32)

    @pl.when(pid == (N // _PB) - 1)
    def _head():
        z = pooled_ref[...]
        for j in range(_NHEAD):
            z = jnp.dot(z, w_ref[j], preferred_element_type=jnp.float32) \
                + b_ref[j]
            if _RELU_AFTER[j]:
                z = jnp.maximum(z, 0.0)
        out_ref[...] = z
        d = z[:, 0:1] - y_ref[...]
        loss_ref[...] = jnp.sqrt(jnp.sum(d * d) * (1.0 / B)).reshape(1, 1)


def _pool_head(batch3, h, wstack, bstack, y):
    return pl.pallas_call(
        _pool_head_body,
        grid=(N // _PB,),
        in_specs=[
            pl.BlockSpec((1, 1, _PB), lambda i: (i, 0, 0)),
            pl.BlockSpec((_PB, D), lambda i: (i, 0)),
            pl.BlockSpec((_NHEAD, D, D), lambda i: (0, 0, 0)),
            pl.BlockSpec((_NHEAD, 1, D), lambda i: (0, 0, 0)),
            pl.BlockSpec((B, 1), lambda i: (0, 0)),
        ],
        out_specs=[
            pl.BlockSpec((B, D), lambda i: (0, 0)),
            pl.BlockSpec((1, 1), lambda i: (0, 0)),
        ],
        out_shape=[
            jax.ShapeDtypeStruct((B, D), jnp.float32),
            jax.ShapeDtypeStruct((1, 1), jnp.float32),
        ],
        scratch_shapes=[pltpu.VMEM((B, D), jnp.float32)],
    )(batch3, h, wstack, bstack, y)


def _pad_w(w):
    out = jnp.zeros((D, D), jnp.float32)
    return out.at[: w.shape[0], : w.shape[1]].set(w)


def _pad_b(b):
    out = jnp.zeros((1, D), jnp.float32)
    return out.at[0, : b.shape[0]].set(b)


def kernel(x, edge_index, edge_attr, batch, y, params):
    src2 = edge_index[0].reshape(E // D, D).astype(jnp.int32)
    ety2 = edge_attr.reshape(E // D, D).astype(jnp.int32)
    gidx = _gidx(src2, ety2).reshape(E)
    dst = edge_index[1].reshape(NW, NCH, CH).astype(jnp.int32)
    zeros = jnp.zeros((NPAD, D), jnp.float32)

    h = x
    for i in range(NUM_LAYERS):
        wrel = params["conv%d_Wrel" % i]
        wcat = wrel.transpose(1, 0, 2).reshape(D, R * D)
        hflat = _mm512(h, wcat).reshape(N * R, D)
        parts = _sc_scatter(hflat, gidx, dst, zeros)
        agg0 = parts[:N]
        agg1 = parts[NPAD:NPAD + N]
        bias = params["conv%d_b" % i].reshape(1, D)
        h = _combine(h, agg0, agg1, params["conv%d_Wroot" % i], bias,
                     relu=(i != NUM_LAYERS - 1))

    wstack = jnp.stack([
        _pad_w(params["fc1_W"]), _pad_w(params["fc2_W"]),
        _pad_w(params["first_W"]),
        _pad_w(params["mlp0_W"]), _pad_w(params["mlp1_W"]),
        _pad_w(params["mlp2_W"]), _pad_w(params["mlp3_W"]),
    ])
    bstack = jnp.stack([
        _pad_b(params["fc1_b"]), _pad_b(params["fc2_b"]),
        _pad_b(params["first_b"]),
        _pad_b(params["mlp0_b"]), _pad_b(params["mlp1_b"]),
        _pad_b(params["mlp2_b"]), _pad_b(params["mlp3_b"]),
    ])
    batch3 = batch.astype(jnp.int32).reshape(N // _PB, 1, _PB)

    out_pad, loss2 = _pool_head(batch3, h, wstack, bstack, y)
    out = out_pad[:, :1]
    loss = loss2[0, 0]
    return (out, loss, loss)


# speedup vs baseline: 9.8360x; 1.0234x over previous
import functools

import jax
import jax.numpy as jnp
from jax import lax
from jax.experimental import pallas as pl
from jax.experimental.pallas import tpu as pltpu
from jax.experimental.pallas import tpu_sc as plsc

N = 10000
E = 320000
D = 128
R = 4
NUM_LAYERS = 3
B = 64

NC = 2
NS = 16
NW = NC * NS
LANES = 16

CH = 80
EW = E // NW
NCH = EW // CH
RT = 632
NPAD = NS * RT


def _sc_scatter_body(h_hbm, gidx_hbm, dst_hbm, zeros_hbm, out_hbm,
                     gidxb, dstb, rows0, rows1, acc, sem0, sem1):
    cid = lax.axis_index("c")
    sid = lax.axis_index("s")
    wid = cid * NS + sid

    pltpu.sync_copy(zeros_hbm.at[pl.ds(sid * RT, RT)],
                    acc.at[pl.ds(sid * RT, RT)])

    pltpu.sync_copy(gidx_hbm.at[pl.ds(wid * EW, EW)], gidxb)
    pltpu.sync_copy(dst_hbm.at[wid], dstb)

    plsc.subcore_barrier()

    def start_gather(ch, buf, sem):
        return pltpu.async_copy(
            h_hbm.at[gidxb.at[pl.ds(ch * CH, CH)]], buf, sem)

    start_gather(0, rows0, sem0)

    def pair_body(k, _):
        ch = 2 * k
        pltpu.make_async_copy(h_hbm.at[gidxb.at[pl.ds(ch * CH, CH)]],
                              rows0, sem0).wait()
        start_gather(ch + 1, rows1, sem1)
        pltpu.sync_copy(rows0, acc.at[dstb.at[ch]], add=True)
        pltpu.make_async_copy(h_hbm.at[gidxb.at[pl.ds((ch + 1) * CH, CH)]],
                              rows1, sem1).wait()
        start_gather(ch + 2, rows0, sem0)
        pltpu.sync_copy(rows1, acc.at[dstb.at[ch + 1]], add=True)
        return 0
    lax.fori_loop(0, (NCH - 1) // 2, pair_body, 0)

    ch_last = NCH - 1
    pltpu.make_async_copy(h_hbm.at[gidxb.at[pl.ds(ch_last * CH, CH)]],
                          rows0, sem0).wait()
    pltpu.sync_copy(rows0, acc.at[dstb.at[ch_last]], add=True)

    plsc.subcore_barrier()
    pltpu.sync_copy(acc.at[pl.ds(sid * RT, RT)],
                    out_hbm.at[pl.ds(cid * NPAD + sid * RT, RT)])


_sc_scatter_cache = []


def _sc_scatter(hflat, gidx, dst, zeros):
    if not _sc_scatter_cache:
        mesh = plsc.VectorSubcoreMesh(core_axis_name="c",
                                      subcore_axis_name="s")
        _sc_scatter_cache.append(pl.kernel(
            _sc_scatter_body,
            out_type=jax.ShapeDtypeStruct((NC * NPAD, D), jnp.float32),
            mesh=mesh,
            scratch_types=[
                pltpu.VMEM((EW,), jnp.int32),
                pltpu.VMEM((NCH, CH), jnp.int32),
                pltpu.VMEM((CH, D), jnp.float32),
                pltpu.VMEM((CH, D), jnp.float32),
                pltpu.VMEM_SHARED((NPAD, D), jnp.float32),
                pltpu.SemaphoreType.DMA,
                pltpu.SemaphoreType.DMA,
            ],
        ))
    return _sc_scatter_cache[0](hflat, gidx, dst, zeros)


_MB = 2000


def _gidx_body(s_ref, t_ref, o_ref):
    o_ref[...] = s_ref[...] * R + t_ref[...]


def _gidx(src2, ety2):
    return pl.pallas_call(
        _gidx_body,
        out_shape=jax.ShapeDtypeStruct((E // D, D), jnp.int32),
    )(src2, ety2)


def _mm512_body(h_ref, w_ref, o_ref):
    o_ref[...] = jnp.dot(h_ref[...], w_ref[...],
                         preferred_element_type=jnp.float32)


def _mm512(h, wcat):
    return pl.pallas_call(
        _mm512_body,
        grid=(N // _MB,),
        in_specs=[
            pl.BlockSpec((_MB, D), lambda i: (i, 0)),
            pl.BlockSpec((D, R * D), lambda i: (0, 0)),
        ],
        out_specs=pl.BlockSpec((_MB, R * D), lambda i: (i, 0)),
        out_shape=jax.ShapeDtypeStruct((N, R * D), jnp.float32),
    )(h, wcat)


def _combine_mm_body(h_ref, a0_ref, a1_ref, w_ref, b_ref, wc_ref,
                     hn_ref, hf_ref):
    z = (jnp.dot(h_ref[...], w_ref[...], preferred_element_type=jnp.float32)
         + a0_ref[...] + a1_ref[...] + b_ref[...])
    z = jnp.maximum(z, 0.0)
    hn_ref[...] = z
    hf_ref[...] = jnp.dot(z, wc_ref[...], preferred_element_type=jnp.float32)


def _combine_mm(h, agg0, agg1, wroot, bias, wcat_next):
    return pl.pallas_call(
        _combine_mm_body,
        grid=(N // _MB,),
        in_specs=[
            pl.BlockSpec((_MB, D), lambda i: (i, 0)),
            pl.BlockSpec((_MB, D), lambda i: (i, 0)),
            pl.BlockSpec((_MB, D), lambda i: (i, 0)),
            pl.BlockSpec((D, D), lambda i: (0, 0)),
            pl.BlockSpec((1, D), lambda i: (0, 0)),
            pl.BlockSpec((D, R * D), lambda i: (0, 0)),
        ],
        out_specs=[
            pl.BlockSpec((_MB, D), lambda i: (i, 0)),
            pl.BlockSpec((_MB, R * D), lambda i: (i, 0)),
        ],
        out_shape=[
            jax.ShapeDtypeStruct((N, D), jnp.float32),
            jax.ShapeDtypeStruct((N, R * D), jnp.float32),
        ],
    )(h, agg0, agg1, wroot, bias, wcat_next)


_PB = 2000
_NHEAD = 7
_RELU_AFTER = (True, False, False, True, True, True, False)


def _pool_head_body(batch_ref, h_ref, a0_ref, a1_ref, wr_ref, br_ref,
                    w_ref, b_ref, y_ref, out_ref, loss_ref, pooled_ref):
    pid = pl.program_id(0)

    @pl.when(pid == 0)
    def _init():
        pooled_ref[...] = jnp.zeros_like(pooled_ref)

    hb = (jnp.dot(h_ref[...], wr_ref[...], preferred_element_type=jnp.float32)
          + a0_ref[...] + a1_ref[...] + br_ref[...])
    bb = batch_ref[...].reshape(1, _PB)
    seg = jax.lax.broadcasted_iota(jnp.int32, (B, _PB), 0)
    onehot = (seg == bb).astype(jnp.float32)
    pooled_ref[...] += jnp.dot(onehot, hb, preferred_element_type=jnp.float32)

    @pl.when(pid == (N // _PB) - 1)
    def _head():
        z = pooled_ref[...]
        for j in range(_NHEAD):
            z = jnp.dot(z, w_ref[j], preferred_element_type=jnp.float32) \
                + b_ref[j]
            if _RELU_AFTER[j]:
                z = jnp.maximum(z, 0.0)
        out_ref[...] = z
        d = z[:, 0:1] - y_ref[...]
        loss_ref[...] = jnp.sqrt(jnp.sum(d * d) * (1.0 / B)).reshape(1, 1)


def _pool_head(batch3, h, agg0, agg1, wroot, bias, wstack, bstack, y):
    return pl.pallas_call(
        _pool_head_body,
        grid=(N // _PB,),
        in_specs=[
            pl.BlockSpec((1, 1, _PB), lambda i: (i, 0, 0)),
            pl.BlockSpec((_PB, D), lambda i: (i, 0)),
            pl.BlockSpec((_PB, D), lambda i: (i, 0)),
            pl.BlockSpec((_PB, D), lambda i: (i, 0)),
            pl.BlockSpec((D, D), lambda i: (0, 0)),
            pl.BlockSpec((1, D), lambda i: (0, 0)),
            pl.BlockSpec((_NHEAD, D, D), lambda i: (0, 0, 0)),
            pl.BlockSpec((_NHEAD, 1, D), lambda i: (0, 0, 0)),
            pl.BlockSpec((B, 1), lambda i: (0, 0)),
        ],
        out_specs=[
            pl.BlockSpec((B, D), lambda i: (0, 0)),
            pl.BlockSpec((1, 1), lambda i: (0, 0)),
        ],
        out_shape=[
            jax.ShapeDtypeStruct((B, D), jnp.float32),
            jax.ShapeDtypeStruct((1, 1), jnp.float32),
        ],
        scratch_shapes=[pltpu.VMEM((B, D), jnp.float32)],
    )(batch3, h, agg0, agg1, wroot, bias, wstack, bstack, y)


def _pad_w(w):
    out = jnp.zeros((D, D), jnp.float32)
    return out.at[: w.shape[0], : w.shape[1]].set(w)


def _pad_b(b):
    out = jnp.zeros((1, D), jnp.float32)
    return out.at[0, : b.shape[0]].set(b)


def kernel(x, edge_index, edge_attr, batch, y, params):
    src2 = edge_index[0].reshape(E // D, D).astype(jnp.int32)
    ety2 = edge_attr.reshape(E // D, D).astype(jnp.int32)
    gidx = _gidx(src2, ety2).reshape(E)
    dst = edge_index[1].reshape(NW, NCH, CH).astype(jnp.int32)
    zeros = jnp.zeros((NPAD, D), jnp.float32)

    wcats = [params["conv%d_Wrel" % i].transpose(1, 0, 2).reshape(D, R * D)
             for i in range(NUM_LAYERS)]
    biases = [params["conv%d_b" % i].reshape(1, D)
              for i in range(NUM_LAYERS)]
    wroots = [params["conv%d_Wroot" % i] for i in range(NUM_LAYERS)]

    h = x
    hflat = _mm512(h, wcats[0]).reshape(N * R, D)
    for i in range(NUM_LAYERS - 1):
        parts = _sc_scatter(hflat, gidx, dst, zeros)
        h, hflat2 = _combine_mm(h, parts[:N], parts[NPAD:NPAD + N],
                                wroots[i], biases[i], wcats[i + 1])
        hflat = hflat2.reshape(N * R, D)
    parts = _sc_scatter(hflat, gidx, dst, zeros)

    wstack = jnp.stack([
        _pad_w(params["fc1_W"]), _pad_w(params["fc2_W"]),
        _pad_w(params["first_W"]),
        _pad_w(params["mlp0_W"]), _pad_w(params["mlp1_W"]),
        _pad_w(params["mlp2_W"]), _pad_w(params["mlp3_W"]),
    ])
    bstack = jnp.stack([
        _pad_b(params["fc1_b"]), _pad_b(params["fc2_b"]),
        _pad_b(params["first_b"]),
        _pad_b(params["mlp0_b"]), _pad_b(params["mlp1_b"]),
        _pad_b(params["mlp2_b"]), _pad_b(params["mlp3_b"]),
    ])
    batch3 = batch.astype(jnp.int32).reshape(N // _PB, 1, _PB)

    out_pad, loss2 = _pool_head(batch3, h, parts[:N], parts[NPAD:NPAD + N],
                                wroots[2], biases[2], wstack, bstack, y)
    out = out_pad[:, :1]
    loss = loss2[0, 0]
    return (out, loss, loss)
